# baseline (speedup 1.0000x reference)
import jax
import jax.numpy as jnp
from jax.experimental import pallas as pl
from jax.experimental.pallas import tpu as pltpu

_N = 1024
_B = 128
_NBLK = _N // _B


def _leaky_relu(x):
    return jnp.maximum(x, 0.2 * x)


def _elu(x):
    return jnp.where(x > 0, x, jnp.exp(jnp.minimum(x, 0.0)) - 1.0)


def _gat_model_kernel(adj_ref, X_ref, W_in_ref, b_in_ref,
                      g0_W_ref, g0_as_ref, g0_ad_ref, g0_b_ref,
                      g1_W_ref, g1_as_ref, g1_ad_ref, g1_b_ref,
                      W_mlp_ref, b_mlp_ref, out_ref,
                      h_ref, ht_ref, ascol_ref, x1_ref):
    k = pl.program_id(0)

    @pl.when(k == 0)
    def _layer0_setup():
        x = jnp.dot(X_ref[...], W_in_ref[...],
                    preferred_element_type=jnp.float32) + b_in_ref[...]
        h = jnp.dot(x, g0_W_ref[...], preferred_element_type=jnp.float32)
        h_ref[...] = h
        ht_ref[...] = h.T
        ascol_ref[...] = jnp.dot(h, g0_as_ref[...].T,
                                 preferred_element_type=jnp.float32)

    @pl.when(k == _NBLK)
    def _layer1_setup():
        h = jnp.dot(x1_ref[...], g1_W_ref[...],
                    preferred_element_type=jnp.float32)
        h_ref[...] = h
        ht_ref[...] = h.T
        ascol_ref[...] = jnp.dot(h, g1_as_ref[...].T,
                                 preferred_element_type=jnp.float32)

    phase0 = k < _NBLK
    jb = jnp.where(phase0, k, k - _NBLK) * _B

    a_dst = jnp.where(phase0, g0_ad_ref[...], g1_ad_ref[...])
    h_blk = h_ref[pl.ds(jb, _B), :]
    ad_b = jnp.dot(h_blk, a_dst.T, preferred_element_type=jnp.float32)
    as_b = ascol_ref[pl.ds(jb, _B), :]
    ed_col = _leaky_relu(as_b + ad_b)
    ad_row = ad_b.T

    mask = adj_ref[...] > 0.0
    s = ascol_ref[...] + ad_row
    e = _leaky_relu(s)
    t = jnp.where(mask, e, -1e30)
    m = jnp.maximum(jnp.max(t, axis=0), ed_col.T[0])
    q = jnp.exp(t - m[None, :])
    d_row = jnp.exp(ed_col.T[0] - m)
    den_row = jnp.sum(q, axis=0) + d_row

    agg = jnp.dot(ht_ref[...], q, preferred_element_type=jnp.float32).T
    d_col = d_row[None, :].T
    den_col = den_row[None, :].T
    out_blk = (agg + d_col * h_blk) / (den_col + 1e-16)

    @pl.when(phase0)
    def _store_layer0():
        x1_ref[pl.ds(jb, _B), :] = _elu(out_blk + g0_b_ref[...])

    @pl.when(jnp.logical_not(phase0))
    def _store_final():
        x2 = _elu(out_blk + g1_b_ref[...])
        o = jnp.dot(x2, W_mlp_ref[...],
                    preferred_element_type=jnp.float32) + b_mlp_ref[...]
        o = jnp.exp(o - jnp.max(o, axis=1, keepdims=True))
        out_ref[...] = o / jnp.sum(o, axis=1, keepdims=True)


def kernel(X, adj, W_in, b_in, g0_W, g0_att_src, g0_att_dst, g0_b,
           g1_W, g1_att_src, g1_att_dst, g1_b, W_mlp, b_mlp):
    N = X.shape[0]
    D_in = X.shape[1]
    C = W_in.shape[1]
    D_out = W_mlp.shape[1]
    v = lambda a: a.reshape(1, -1)
    full = lambda shape: pl.BlockSpec(shape, lambda k: (0, 0))
    return pl.pallas_call(
        _gat_model_kernel,
        grid=(2 * _NBLK,),
        in_specs=[
            pl.BlockSpec((N, _B), lambda k: (0, jax.lax.rem(k, _NBLK))),
            full((N, D_in)), full((D_in, C)), full((1, C)),
            full((C, C)), full((1, C)), full((1, C)), full((1, C)),
            full((C, C)), full((1, C)), full((1, C)), full((1, C)),
            full((C, D_out)), full((1, D_out)),
        ],
        out_specs=pl.BlockSpec(
            (_B, D_out), lambda k: (jnp.maximum(k - _NBLK, 0), 0)),
        out_shape=jax.ShapeDtypeStruct((N, D_out), jnp.float32),
        scratch_shapes=[
            pltpu.VMEM((N, C), jnp.float32),
            pltpu.VMEM((C, N), jnp.float32),
            pltpu.VMEM((N, 1), jnp.float32),
            pltpu.VMEM((N, C), jnp.float32),
        ],
        compiler_params=pltpu.CompilerParams(
            dimension_semantics=("arbitrary",)),
    )(adj, X, W_in, v(b_in),
      g0_W, v(g0_att_src), v(g0_att_dst), v(g0_b),
      g1_W, v(g1_att_src), v(g1_att_dst), v(g1_b),
      W_mlp, v(b_mlp))


# speedup vs baseline: 1.8980x; 1.8980x over previous
import jax
import jax.numpy as jnp
from jax.experimental import pallas as pl


def _leaky_relu(x):
    return jnp.maximum(x, 0.2 * x)


def _elu(x):
    return jnp.where(x > 0, x, jnp.exp(jnp.minimum(x, 0.0)) - 1.0)


def _gat_model_kernel(adj_ref, X_ref, W_in_ref, b_in_ref,
                      g0_W_ref, g0_as_ref, g0_ad_ref, g0_b_ref,
                      g1_W_ref, g1_as_ref, g1_ad_ref, g1_b_ref,
                      W_mlp_ref, b_mlp_ref, out_ref):
    mask = adj_ref[...] > 0.0

    def gat_t(h_t, as_col, ad_row, as_row, b_col):
        s = as_col + ad_row
        e = _leaky_relu(s)
        t = jnp.where(mask, e, -1e30)
        ed = _leaky_relu(as_row + ad_row)
        m = jnp.maximum(jnp.max(t, axis=0, keepdims=True), ed)
        q = jnp.exp(t - m)
        d = jnp.exp(ed - m)
        den = jnp.sum(q, axis=0, keepdims=True) + d
        agg = jnp.dot(h_t, q, preferred_element_type=jnp.float32)
        return (agg + d * h_t) / (den + 1e-16) + b_col

    x = jnp.dot(X_ref[...], W_in_ref[...],
                preferred_element_type=jnp.float32) + b_in_ref[...]
    h0 = jnp.dot(x, g0_W_ref[...], preferred_element_type=jnp.float32)
    h0_t = h0.T
    as_col0 = jnp.dot(h0, g0_as_ref[...].T,
                      preferred_element_type=jnp.float32)
    ad_row0 = jnp.dot(g0_ad_ref[...], h0_t,
                      preferred_element_type=jnp.float32)
    as_row0 = jnp.dot(g0_as_ref[...], h0_t,
                      preferred_element_type=jnp.float32)
    x1_t = _elu(gat_t(h0_t, as_col0, ad_row0, as_row0,
                      g0_b_ref[...].T))

    h1_t = jnp.dot(g1_W_ref[...].T, x1_t,
                   preferred_element_type=jnp.float32)
    as_row1 = jnp.dot(g1_as_ref[...], h1_t,
                      preferred_element_type=jnp.float32)
    ad_row1 = jnp.dot(g1_ad_ref[...], h1_t,
                      preferred_element_type=jnp.float32)
    as_col1 = as_row1.T
    x2_t = _elu(gat_t(h1_t, as_col1, ad_row1, as_row1,
                      g1_b_ref[...].T))

    o_t = jnp.dot(W_mlp_ref[...].T, x2_t,
                  preferred_element_type=jnp.float32) + b_mlp_ref[...].T
    o_t = jnp.exp(o_t - jnp.max(o_t, axis=0, keepdims=True))
    o_t = o_t / jnp.sum(o_t, axis=0, keepdims=True)
    out_ref[...] = o_t.T


def kernel(X, adj, W_in, b_in, g0_W, g0_att_src, g0_att_dst, g0_b,
           g1_W, g1_att_src, g1_att_dst, g1_b, W_mlp, b_mlp):
    N = X.shape[0]
    D_out = W_mlp.shape[1]
    v = lambda a: a.reshape(1, -1)
    return pl.pallas_call(
        _gat_model_kernel,
        out_shape=jax.ShapeDtypeStruct((N, D_out), jnp.float32),
    )(adj, X, W_in, v(b_in),
      g0_W, v(g0_att_src), v(g0_att_dst), v(g0_b),
      g1_W, v(g1_att_src), v(g1_att_dst), v(g1_b),
      W_mlp, v(b_mlp))
